# Initial kernel scaffold; baseline (speedup 1.0000x reference)
import jax
import jax.numpy as jnp
from jax.experimental import pallas as pl


def kernel(hidden_states, cos, sin, cu_seqlens, W_qkvw, W_o):
    raise NotImplementedError("write your pallas kernel here")


# speedup vs baseline: 2.6375x; 2.6375x over previous
import functools
import math

import jax
import jax.numpy as jnp
from jax.experimental import pallas as pl
from jax.experimental.pallas import tpu as pltpu

HID = 2048
NH = 16
NKV = 4
G = NH // NKV
DQK = 128
DV = 128
S = 2048
BLOCK = 32
TOPN = 8
WIN = 512
T = S // BLOCK
OUT_DIM = 3120
N_ROPE_TILES = (NH + NKV)

_SCALE = 1.0 / math.sqrt(DQK)


def _proj_rope_body(x_ref, w_ref, cos_ref, sin_ref, o_ref):
    j = pl.program_id(1)
    acc = jax.lax.dot_general(
        x_ref[...], w_ref[...], (((1,), (1,)), ((), ())),
        preferred_element_type=jnp.float32)
    c = cos_ref[...]
    s = sin_ref[...]
    a1 = acc[:, : DQK // 2]
    a2 = acc[:, DQK // 2:]
    rot = jnp.concatenate([-a2, a1], axis=1)
    roped = acc * c + rot * s
    o_ref[...] = jnp.where(j < N_ROPE_TILES, roped, acc)


def _proj_rope(x, w_qkvw, cos, sin):
    TS = 512
    grid = (S // TS, pl.cdiv(OUT_DIM, DQK))
    return pl.pallas_call(
        _proj_rope_body,
        grid=grid,
        in_specs=[
            pl.BlockSpec((TS, HID), lambda i, j: (i, 0)),
            pl.BlockSpec((DQK, HID), lambda i, j: (j, 0)),
            pl.BlockSpec((TS, DQK), lambda i, j: (i, 0)),
            pl.BlockSpec((TS, DQK), lambda i, j: (i, 0)),
        ],
        out_specs=pl.BlockSpec((TS, DQK), lambda i, j: (i, j)),
        out_shape=jax.ShapeDtypeStruct((S, OUT_DIM), jnp.float32),
    )(x, w_qkvw, cos, sin)


QB = 128
R = QB * G


def _attn_body(q_ref, k_ref, v_ref, w_ref, o_ref):
    qi = pl.program_id(1)
    q0 = qi * QB

    qg = q_ref[...].reshape(R, DQK)
    k = k_ref[0]
    v = v_ref[0]

    ii = q0 + jax.lax.broadcasted_iota(jnp.int32, (R, S), 0) // G
    jj = jax.lax.broadcasted_iota(jnp.int32, (R, S), 1)

    kc = k.reshape(T, BLOCK, DQK).mean(axis=1)
    vc = v.reshape(T, BLOCK, DV).mean(axis=1)
    s_cmp = jax.lax.dot_general(
        qg, kc, (((1,), (1,)), ((), ())),
        preferred_element_type=jnp.float32) * _SCALE
    iic = q0 + jax.lax.broadcasted_iota(jnp.int32, (R, T), 0) // G
    jbc = jax.lax.broadcasted_iota(jnp.int32, (R, T), 1)
    s_cmp = jnp.where(jbc * BLOCK <= iic, s_cmp, -1e9)
    m = jnp.max(s_cmp, axis=1, keepdims=True)
    p_cmp = jnp.exp(s_cmp - m)
    p_cmp = p_cmp / jnp.sum(p_cmp, axis=1, keepdims=True)
    o_cmp = jnp.dot(p_cmp, vc, preferred_element_type=jnp.float32)

    imp = p_cmp.reshape(QB, G, T).sum(axis=1)
    iota_t = jax.lax.broadcasted_iota(jnp.int32, (QB, T), 1)
    kbidx = jax.lax.broadcasted_iota(jnp.int32, (QB, S), 1) // BLOCK
    selk = jnp.zeros((QB, S), dtype=jnp.bool_)
    work = imp
    for _ in range(TOPN):
        mx = jnp.max(work, axis=1, keepdims=True)
        cand = jnp.where(work == mx, iota_t, T)
        js = jnp.min(cand, axis=1, keepdims=True)
        selk = selk | (kbidx == js)
        work = jnp.where(iota_t == js, -jnp.inf, work)
    sel_rows = jnp.broadcast_to(selk[:, None, :], (QB, G, S)).reshape(R, S)

    s_full = jax.lax.dot_general(
        qg, k, (((1,), (1,)), ((), ())),
        preferred_element_type=jnp.float32) * _SCALE
    causal = jj <= ii

    def _softmax_pv(scores):
        mloc = jnp.max(scores, axis=1, keepdims=True)
        p = jnp.exp(scores - mloc)
        p = p / jnp.sum(p, axis=1, keepdims=True)
        return jnp.dot(p, v, preferred_element_type=jnp.float32)

    o_sel = _softmax_pv(jnp.where(sel_rows & causal, s_full, -1e9))
    o_win = _softmax_pv(jnp.where(causal & (ii - jj < WIN), s_full, -1e9))

    g = jax.nn.sigmoid(w_ref[...].reshape(R, 3))
    out = (g[:, 0:1] * o_cmp + g[:, 1:2] * o_sel + g[:, 2:3] * o_win)
    o_ref[...] = out.reshape(QB, 1, G, DV)


def _attn(q, k, v, w):
    grid = (NKV, S // QB)
    return pl.pallas_call(
        _attn_body,
        grid=grid,
        in_specs=[
            pl.BlockSpec((QB, 1, G, DQK), lambda h, i: (i, h, 0, 0)),
            pl.BlockSpec((1, S, DQK), lambda h, i: (h, 0, 0)),
            pl.BlockSpec((1, S, DV), lambda h, i: (h, 0, 0)),
            pl.BlockSpec((QB, 1, G, 3), lambda h, i: (i, h, 0, 0)),
        ],
        out_specs=pl.BlockSpec((QB, 1, G, DV), lambda h, i: (i, h, 0, 0)),
        out_shape=jax.ShapeDtypeStruct((S, NKV, G, DV), jnp.float32),
    )(q, k, v, w)


def _out_proj_body(x_ref, w_ref, o_ref):
    o_ref[...] = jax.lax.dot_general(
        x_ref[...], w_ref[...], (((1,), (1,)), ((), ())),
        preferred_element_type=jnp.float32)


def _out_proj(x, w_o):
    TS = 512
    TN = 512
    grid = (S // TS, HID // TN)
    return pl.pallas_call(
        _out_proj_body,
        grid=grid,
        in_specs=[
            pl.BlockSpec((TS, NH * DV), lambda i, j: (i, 0)),
            pl.BlockSpec((TN, NH * DV), lambda i, j: (j, 0)),
        ],
        out_specs=pl.BlockSpec((TS, TN), lambda i, j: (i, j)),
        out_shape=jax.ShapeDtypeStruct((S, HID), jnp.float32),
    )(x, w_o)


def kernel(hidden_states, cos, sin, cu_seqlens, W_qkvw, W_o):
    s, b, d = hidden_states.shape
    x = hidden_states.reshape(s, d)
    qkvw = _proj_rope(x, W_qkvw, cos, sin)
    o0 = NH * DQK
    o1 = o0 + NKV * DQK
    o2 = o1 + NKV * DV
    q = qkvw[:, :o0].reshape(S, NKV, G, DQK)
    k = qkvw[:, o0:o1].reshape(S, NKV, DQK).transpose(1, 0, 2)
    v = qkvw[:, o1:o2].reshape(S, NKV, DV).transpose(1, 0, 2)
    w = qkvw[:, o2:].reshape(S, NKV, G, 3)
    o = _attn(q, k, v, w)
    out = _out_proj(o.reshape(S, NH * DV), W_o)
    return out.reshape(s, b, HID)
